# baseline (speedup 1.0000x reference)
import functools

import jax
import jax.numpy as jnp
from jax import lax
from jax.experimental import pallas as pl
from jax.experimental.pallas import tpu as pltpu
from jax.experimental.pallas import tpu_sc as plsc

N_NODES = 10000
DIM = 128
N_EDGES = 320000
LANES = 16

NUM_CORES = 2
NUM_SUBCORES = 16
NUM_WORKERS = NUM_CORES * NUM_SUBCORES
EDGES_PER_WORKER = N_EDGES // NUM_WORKERS
CHUNK = 80
NUM_CHUNKS = EDGES_PER_WORKER // CHUNK
N_PAD = 10240
ROWS_PER_TILE = N_PAD // NUM_SUBCORES
ZROWS = 128

_MESH = plsc.VectorSubcoreMesh(
    core_axis_name="c", subcore_axis_name="s",
    num_cores=NUM_CORES, num_subcores=NUM_SUBCORES)


def _broadcast_lane(val16, k):
    return lax.gather(
        val16, jnp.full((LANES, 1), k, jnp.int32),
        dimension_numbers=lax.GatherDimensionNumbers(
            offset_dims=(), collapsed_slice_dims=(0,),
            start_index_map=(0,)),
        slice_sizes=(1,),
        mode=lax.GatherScatterMode.PROMISE_IN_BOUNDS)


NBUF = 3


def _propagate_body(h, rv3, col, out, acc,
                    rows0, rows1, rows2, rvv0, rvv1, rvv2,
                    rowv0, rowv1, rowv2, colf,
                    gsem0, gsem1, gsem2, ssem0, ssem1, ssem2,
                    rsem0, rsem1, rsem2, psem):
    c = lax.axis_index("c")
    s = lax.axis_index("s")
    wid = c * NUM_SUBCORES + s
    ebase = wid * EDGES_PER_WORKER

    rows = (rows0, rows1, rows2)
    rvv = (rvv0, rvv1, rvv2)
    rowv = (rowv0, rowv1, rowv2)
    gsem = (gsem0, gsem1, gsem2)
    ssem = (ssem0, ssem1, ssem2)
    rsem = (rsem0, rsem1, rsem2)

    def issue_gather(i, b):
        pltpu.async_copy(h.at[colf.at[pl.ds(i * CHUNK, CHUNK)]], rows[b],
                         gsem[b])

    def wait_gather(b):
        pltpu.make_async_copy(h.at[colf.at[pl.ds(0, CHUNK)]], rows[b],
                              gsem[b]).wait()

    def issue_rowval(i, b):
        pltpu.async_copy(rv3.at[wid * NUM_CHUNKS + i], rvv[b], rsem[b])

    def wait_rowval(b):
        pltpu.make_async_copy(rv3.at[0], rvv[b], rsem[b]).wait()

    def issue_scatter(b):
        pltpu.async_copy(rows[b], acc.at[rowv[b]], ssem[b], add=True)

    def wait_scatter(b):
        pltpu.make_async_copy(rows[b], acc.at[rowv0], ssem[b]).wait()

    def unpack_rows(b):
        for g in range(CHUNK // LANES):
            sl = pl.ds(g * LANES, LANES)
            rowv[b][sl] = rvv[b][0, sl].astype(jnp.int32)

    def scale(b):
        def scale_body(g, icarry):
            val16 = rvv[b][1, pl.ds(g * LANES, LANES)]
            for k in range(LANES):
                vb = _broadcast_lane(val16, k)
                e = g * LANES + k
                for j in range(DIM // LANES):
                    sl = pl.ds(j * LANES, LANES)
                    rows[b][e, sl] = rows[b][e, sl] * vb
            return icarry
        lax.fori_loop(0, CHUNK // LANES, scale_body, 0)

    pltpu.async_copy(col.at[pl.ds(ebase, EDGES_PER_WORKER)], colf, psem)
    for b in range(NBUF):
        issue_rowval(b, b)

    def zrow(i, carry):
        for j in range(DIM // LANES):
            rows0[i, pl.ds(j * LANES, LANES)] = jnp.zeros((LANES,),
                                                          jnp.float32)
        return carry
    lax.fori_loop(0, CHUNK, zrow, 0)
    nbase = s * ROWS_PER_TILE
    for z in range(ROWS_PER_TILE // CHUNK):
        pltpu.sync_copy(rows0, acc.at[pl.ds(nbase + z * CHUNK, CHUNK)])

    pltpu.make_async_copy(col.at[pl.ds(0, EDGES_PER_WORKER)], colf,
                          psem).wait()
    plsc.subcore_barrier()

    issue_gather(0, 0)
    issue_gather(1, 1)

    wait_gather(0)
    wait_rowval(0)
    unpack_rows(0)
    scale(0)
    issue_scatter(0)
    issue_gather(2, 2)

    def main_body(t, carry):
        for u in range(NBUF):
            i = 1 + NBUF * t + u
            b = (1 + u) % NBUF
            b2 = u
            wait_gather(b)
            wait_rowval(b)
            unpack_rows(b)
            scale(b)
            issue_scatter(b)

            @pl.when(i <= NUM_CHUNKS - 3)
            def _():
                wait_scatter(b2)
                issue_gather(i + 2, b2)
                issue_rowval(i + 2, b2)
        return carry
    lax.fori_loop(0, (NUM_CHUNKS - 2) // NBUF, main_body, 0)

    wait_gather((NUM_CHUNKS - 1) % NBUF)
    wait_rowval((NUM_CHUNKS - 1) % NBUF)
    unpack_rows((NUM_CHUNKS - 1) % NBUF)
    scale((NUM_CHUNKS - 1) % NBUF)
    issue_scatter((NUM_CHUNKS - 1) % NBUF)

    for i in (NUM_CHUNKS - 3, NUM_CHUNKS - 2, NUM_CHUNKS - 1):
        wait_scatter(i % NBUF)
    plsc.subcore_barrier()
    pltpu.sync_copy(acc.at[pl.ds(nbase, ROWS_PER_TILE)],
                    out.at[c].at[pl.ds(nbase, ROWS_PER_TILE)])


_propagate = functools.partial(
    pl.kernel,
    out_type=jax.ShapeDtypeStruct((NUM_CORES, N_PAD, DIM), jnp.float32),
    mesh=_MESH,
    scratch_types=[
        pltpu.VMEM_SHARED((N_PAD, DIM), jnp.float32),
        pltpu.VMEM((CHUNK, DIM), jnp.float32),
        pltpu.VMEM((CHUNK, DIM), jnp.float32),
        pltpu.VMEM((CHUNK, DIM), jnp.float32),
        pltpu.VMEM((2, CHUNK), jnp.float32),
        pltpu.VMEM((2, CHUNK), jnp.float32),
        pltpu.VMEM((2, CHUNK), jnp.float32),
        pltpu.VMEM((CHUNK,), jnp.int32),
        pltpu.VMEM((CHUNK,), jnp.int32),
        pltpu.VMEM((CHUNK,), jnp.int32),
        pltpu.VMEM((EDGES_PER_WORKER,), jnp.int32),
        pltpu.SemaphoreType.DMA,
        pltpu.SemaphoreType.DMA,
        pltpu.SemaphoreType.DMA,
        pltpu.SemaphoreType.DMA,
        pltpu.SemaphoreType.DMA,
        pltpu.SemaphoreType.DMA,
        pltpu.SemaphoreType.DMA,
        pltpu.SemaphoreType.DMA,
        pltpu.SemaphoreType.DMA,
        pltpu.SemaphoreType.DMA,
    ],
)(_propagate_body)


_BLK = 400


def _sum2_body(a_ref, b_ref, o_ref):
    o_ref[...] = a_ref[...] + b_ref[...]


def _tc_sum2(a, b):
    return pl.pallas_call(
        _sum2_body,
        out_shape=jax.ShapeDtypeStruct((N_NODES, DIM), jnp.float32),
        grid=(N_NODES // _BLK,),
        in_specs=[pl.BlockSpec((_BLK, DIM), lambda i: (i, 0))] * 2,
        out_specs=pl.BlockSpec((_BLK, DIM), lambda i: (i, 0)),
    )(a, b)


def _mean4_body(a_ref, b_ref, c_ref, d_ref, o_ref):
    o_ref[...] = (a_ref[...] + b_ref[...] + c_ref[...] + d_ref[...]) * (
        1.0 / 3.0)


def _tc_mean4(a, b, c, d):
    return pl.pallas_call(
        _mean4_body,
        out_shape=jax.ShapeDtypeStruct((N_NODES, DIM), jnp.float32),
        grid=(N_NODES // _BLK,),
        in_specs=[pl.BlockSpec((_BLK, DIM), lambda i: (i, 0))] * 4,
        out_specs=pl.BlockSpec((_BLK, DIM), lambda i: (i, 0)),
    )(a, b, c, d)


def kernel(x, adj_indices, adj_values, keep_rate):
    del keep_rate
    row = adj_indices[0]
    col = adj_indices[1]
    rv3 = jnp.stack([row.astype(jnp.float32).reshape(
                         N_EDGES // CHUNK, CHUNK),
                     adj_values.reshape(N_EDGES // CHUNK, CHUNK)], axis=1)
    p1 = _propagate(x, rv3, col)
    h1 = _tc_sum2(p1[0, :N_NODES], p1[1, :N_NODES])
    p2 = _propagate(h1, rv3, col)
    out = _tc_mean4(x, h1, p2[0, :N_NODES], p2[1, :N_NODES])
    return out


# speedup vs baseline: 1.0690x; 1.0690x over previous
import functools

import jax
import jax.numpy as jnp
from jax import lax
from jax.experimental import pallas as pl
from jax.experimental.pallas import tpu as pltpu
from jax.experimental.pallas import tpu_sc as plsc

N_NODES = 10000
DIM = 128
N_EDGES = 320000
LANES = 16

NUM_CORES = 2
NUM_SUBCORES = 16
NUM_WORKERS = NUM_CORES * NUM_SUBCORES
EDGES_PER_WORKER = N_EDGES // NUM_WORKERS
CHUNK = 80
NUM_CHUNKS = EDGES_PER_WORKER // CHUNK
N_PAD = 10240
ROWS_PER_TILE = N_PAD // NUM_SUBCORES
ZROWS = 128

_MESH = plsc.VectorSubcoreMesh(
    core_axis_name="c", subcore_axis_name="s",
    num_cores=NUM_CORES, num_subcores=NUM_SUBCORES)


def _broadcast_lane(val16, k):
    return lax.gather(
        val16, jnp.full((LANES, 1), k, jnp.int32),
        dimension_numbers=lax.GatherDimensionNumbers(
            offset_dims=(), collapsed_slice_dims=(0,),
            start_index_map=(0,)),
        slice_sizes=(1,),
        mode=lax.GatherScatterMode.PROMISE_IN_BOUNDS)


NBUF = 3


def _propagate_body(h, row, col, val, out, acc,
                    rows0, rows1, rows2, rowv0, rowv1, rowv2,
                    valv0, valv1, valv2, colf,
                    gsem0, gsem1, gsem2, ssem0, ssem1, ssem2,
                    rsem0, rsem1, rsem2, psem):
    c = lax.axis_index("c")
    s = lax.axis_index("s")
    wid = c * NUM_SUBCORES + s
    ebase = wid * EDGES_PER_WORKER

    rows = (rows0, rows1, rows2)
    rowv = (rowv0, rowv1, rowv2)
    valv = (valv0, valv1, valv2)
    gsem = (gsem0, gsem1, gsem2)
    ssem = (ssem0, ssem1, ssem2)
    rsem = (rsem0, rsem1, rsem2)

    def issue_gather(i, b):
        pltpu.async_copy(h.at[colf.at[pl.ds(i * CHUNK, CHUNK)]], rows[b],
                         gsem[b])

    def wait_gather(b):
        pltpu.make_async_copy(h.at[colf.at[pl.ds(0, CHUNK)]], rows[b],
                              gsem[b]).wait()

    def issue_rowval(i, b):
        pltpu.async_copy(row.at[pl.ds(ebase + i * CHUNK, CHUNK)], rowv[b],
                         rsem[b])
        pltpu.async_copy(val.at[pl.ds(ebase + i * CHUNK, CHUNK)], valv[b],
                         rsem[b])

    def wait_rowval(b):
        pltpu.make_async_copy(row.at[pl.ds(0, CHUNK)], rowv[b],
                              rsem[b]).wait()
        pltpu.make_async_copy(val.at[pl.ds(0, CHUNK)], valv[b],
                              rsem[b]).wait()

    def issue_scatter(b):
        pltpu.async_copy(rows[b], acc.at[rowv[b]], ssem[b], add=True)

    def wait_scatter(b):
        pltpu.make_async_copy(rows[b], acc.at[rowv0], ssem[b]).wait()

    def scale(b):
        def scale_body(g, icarry):
            val16 = valv[b][pl.ds(g * LANES, LANES)]
            for k in range(LANES):
                vb = _broadcast_lane(val16, k)
                e = g * LANES + k
                for j in range(DIM // LANES):
                    sl = pl.ds(j * LANES, LANES)
                    rows[b][e, sl] = rows[b][e, sl] * vb
            return icarry
        lax.fori_loop(0, CHUNK // LANES, scale_body, 0)

    pltpu.async_copy(col.at[pl.ds(ebase, EDGES_PER_WORKER)], colf, psem)
    for b in range(NBUF):
        issue_rowval(b, b)

    def zrow(i, carry):
        for j in range(DIM // LANES):
            rows0[i, pl.ds(j * LANES, LANES)] = jnp.zeros((LANES,),
                                                          jnp.float32)
        return carry
    lax.fori_loop(0, CHUNK, zrow, 0)
    nbase = s * ROWS_PER_TILE
    for z in range(ROWS_PER_TILE // CHUNK):
        pltpu.sync_copy(rows0, acc.at[pl.ds(nbase + z * CHUNK, CHUNK)])

    pltpu.make_async_copy(col.at[pl.ds(0, EDGES_PER_WORKER)], colf,
                          psem).wait()
    plsc.subcore_barrier()

    issue_gather(0, 0)
    issue_gather(1, 1)

    wait_gather(0)
    wait_rowval(0)
    scale(0)
    issue_scatter(0)
    issue_gather(2, 2)

    def main_body(t, carry):
        for u in range(NBUF):
            i = 1 + NBUF * t + u
            b = (1 + u) % NBUF
            b2 = u
            wait_gather(b)
            wait_rowval(b)
            scale(b)
            issue_scatter(b)

            @pl.when(i <= NUM_CHUNKS - 3)
            def _():
                wait_scatter(b2)
                issue_gather(i + 2, b2)
                issue_rowval(i + 2, b2)
        return carry
    lax.fori_loop(0, (NUM_CHUNKS - 2) // NBUF, main_body, 0)

    wait_gather((NUM_CHUNKS - 1) % NBUF)
    wait_rowval((NUM_CHUNKS - 1) % NBUF)
    scale((NUM_CHUNKS - 1) % NBUF)
    issue_scatter((NUM_CHUNKS - 1) % NBUF)

    for i in (NUM_CHUNKS - 3, NUM_CHUNKS - 2, NUM_CHUNKS - 1):
        wait_scatter(i % NBUF)
    plsc.subcore_barrier()
    pltpu.sync_copy(acc.at[pl.ds(nbase, ROWS_PER_TILE)],
                    out.at[c].at[pl.ds(nbase, ROWS_PER_TILE)])


_propagate = functools.partial(
    pl.kernel,
    out_type=jax.ShapeDtypeStruct((NUM_CORES, N_PAD, DIM), jnp.float32),
    mesh=_MESH,
    scratch_types=[
        pltpu.VMEM_SHARED((N_PAD, DIM), jnp.float32),
        pltpu.VMEM((CHUNK, DIM), jnp.float32),
        pltpu.VMEM((CHUNK, DIM), jnp.float32),
        pltpu.VMEM((CHUNK, DIM), jnp.float32),
        pltpu.VMEM((CHUNK,), jnp.int32),
        pltpu.VMEM((CHUNK,), jnp.int32),
        pltpu.VMEM((CHUNK,), jnp.int32),
        pltpu.VMEM((CHUNK,), jnp.float32),
        pltpu.VMEM((CHUNK,), jnp.float32),
        pltpu.VMEM((CHUNK,), jnp.float32),
        pltpu.VMEM((EDGES_PER_WORKER,), jnp.int32),
        pltpu.SemaphoreType.DMA,
        pltpu.SemaphoreType.DMA,
        pltpu.SemaphoreType.DMA,
        pltpu.SemaphoreType.DMA,
        pltpu.SemaphoreType.DMA,
        pltpu.SemaphoreType.DMA,
        pltpu.SemaphoreType.DMA,
        pltpu.SemaphoreType.DMA,
        pltpu.SemaphoreType.DMA,
        pltpu.SemaphoreType.DMA,
    ],
)(_propagate_body)


_BLK = 400


def _sum2_body(a_ref, b_ref, o_ref):
    o_ref[...] = a_ref[...] + b_ref[...]


def _tc_sum2(a, b):
    return pl.pallas_call(
        _sum2_body,
        out_shape=jax.ShapeDtypeStruct((N_NODES, DIM), jnp.float32),
        grid=(N_NODES // _BLK,),
        in_specs=[pl.BlockSpec((_BLK, DIM), lambda i: (i, 0))] * 2,
        out_specs=pl.BlockSpec((_BLK, DIM), lambda i: (i, 0)),
    )(a, b)


def _mean4_body(a_ref, b_ref, c_ref, d_ref, o_ref):
    o_ref[...] = (a_ref[...] + b_ref[...] + c_ref[...] + d_ref[...]) * (
        1.0 / 3.0)


def _tc_mean4(a, b, c, d):
    return pl.pallas_call(
        _mean4_body,
        out_shape=jax.ShapeDtypeStruct((N_NODES, DIM), jnp.float32),
        grid=(N_NODES // _BLK,),
        in_specs=[pl.BlockSpec((_BLK, DIM), lambda i: (i, 0))] * 4,
        out_specs=pl.BlockSpec((_BLK, DIM), lambda i: (i, 0)),
    )(a, b, c, d)


def kernel(x, adj_indices, adj_values, keep_rate):
    del keep_rate
    row = adj_indices[0]
    col = adj_indices[1]
    p1 = _propagate(x, row, col, adj_values)
    h1 = _tc_sum2(p1[0, :N_NODES], p1[1, :N_NODES])
    p2 = _propagate(h1, row, col, adj_values)
    out = _tc_mean4(x, h1, p2[0, :N_NODES], p2[1, :N_NODES])
    return out
